# speedup vs baseline: 1.8319x; 1.8319x over previous
import jax
import jax.numpy as jnp
from jax import lax
from jax.experimental import pallas as pl
from jax.experimental.pallas import tpu as pltpu
from jax.experimental.pallas import tpu_sc as plsc

VOCAB = 32000
HIDDEN = 4096
BATCH = 4
SEQ = 4096

NC = 2
NS = 16
NW = NC * NS

B = BATCH * SEQ
B_PER_W = B // NW
K = 8
NBUF = 2
NCHUNK = B_PER_W // K


def _gather_body(ids_hbm, table_hbm, out_hbm, idx_v, bufs, gsems, psems):
    wid = lax.axis_index("s") * NC + lax.axis_index("c")
    base = wid * B_PER_W

    pltpu.sync_copy(ids_hbm.at[pl.ds(base, B_PER_W)], idx_v)

    def gather_start(g, b):
        pltpu.async_copy(table_hbm.at[idx_v.at[pl.ds(g * K, K)]], bufs[b],
                         gsems[b])

    def gather_wait(b):
        pltpu.make_async_copy(table_hbm.at[idx_v.at[pl.ds(0, K)]], bufs[b],
                              gsems[b]).wait()

    for b in range(NBUF):
        gather_start(b, b)

    @pl.loop(0, NCHUNK - NBUF, step=NBUF)
    def _(g0):
        for b in range(NBUF):
            g = g0 + b
            gather_wait(b)
            put = pltpu.async_copy(bufs[b], out_hbm.at[pl.ds(base + g * K, K)],
                                   psems[b])
            put.wait()
            gather_start(g + NBUF, b)

    for b in range(NBUF):
        g = (NCHUNK - NBUF) + b
        gather_wait(b)
        pltpu.async_copy(bufs[b], out_hbm.at[pl.ds(base + g * K, K)],
                         psems[b]).wait()


@jax.jit
def _embed_gather(ids, W):
    mesh = plsc.VectorSubcoreMesh(core_axis_name="c", subcore_axis_name="s")
    run = pl.kernel(
        _gather_body,
        out_type=jax.ShapeDtypeStruct((B, HIDDEN), jnp.float32),
        mesh=mesh,
        scratch_types=[
            pltpu.VMEM((B_PER_W,), jnp.int32),
            [pltpu.VMEM((K, HIDDEN), jnp.float32) for _ in range(NBUF)],
            [pltpu.SemaphoreType.DMA for _ in range(NBUF)],
            [pltpu.SemaphoreType.DMA for _ in range(NBUF)],
        ],
    )
    return run(ids, W)


def kernel(inputs, W):
    ids = inputs[..., 0].reshape(B)
    attention_mask = inputs[..., 1]
    hidden = _embed_gather(ids, W)
    return hidden.reshape(BATCH, SEQ, HIDDEN), attention_mask
